# baseline (speedup 1.0000x reference)
import functools

import jax
import jax.numpy as jnp
from jax import lax
from jax.experimental import pallas as pl
from jax.experimental.pallas import tpu as pltpu, tpu_sc as plsc

D = 32
SEQ = 50
NUM_WORKERS = 32
UNIT_BAGS = 128
NBUF = 2
TNBUF = 4
CBLK = 256


def _transpose_kernel(vocab, t32_hbm, tail_hbm, out_hbm,
                      in0, in1, in2, in3, ob0, ob1, ob2, ob3,
                      sin0, sin1, sin2, sin3, sout0, sout1, sout2, sout3):
    nfull = vocab // CBLK
    rem = vocab - nfull * CBLK
    maxnb = (nfull + NUM_WORKERS - 1) // NUM_WORKERS
    nrounds = (maxnb + TNBUF - 1) // TNBUF

    wid = lax.axis_index("s") * 2 + lax.axis_index("c")
    nb = (nfull - wid + NUM_WORKERS - 1) // NUM_WORKERS

    ins = (in0, in1, in2, in3)
    obs = (ob0, ob1, ob2, ob3)
    sins = (sin0, sin1, sin2, sin3)
    souts = (sout0, sout1, sout2, sout3)

    iota = lax.iota(jnp.int32, 16)

    def fire_in(k, p):
        c0 = (wid + k * NUM_WORKERS) * CBLK
        pltpu.async_copy(t32_hbm.at[:, pl.ds(c0, CBLK)], ins[p], sins[p])

    def transpose(in_ref, ob_ref):
        @plsc.parallel_loop(0, 16, unroll=4)
        def tbody(t):
            dvec = (iota + t) & 15
            svec = iota * D + dvec
            for dg in range(D // 16):
                gd = dvec + dg * 16
                for cg in range(CBLK // 16):
                    v = plsc.load_gather(in_ref, [gd, iota + cg * 16])
                    plsc.store_scatter(
                        ob_ref, [svec + (cg * 16 * D + dg * 16)], v)

    for p in range(TNBUF - 1):
        @pl.when(p < nb)
        def _():
            fire_in(p, p)

    def rnd(kr, carry):
        for p in range(TNBUF):
            k = kr * TNBUF + p

            @pl.when(k < nb)
            def _():
                pltpu.make_async_copy(
                    t32_hbm.at[:, pl.ds(0, CBLK)], ins[p], sins[p]).wait()

                @pl.when(k + TNBUF - 1 < nb)
                def _():
                    fire_in(k + TNBUF - 1, (p + TNBUF - 1) % TNBUF)

                @pl.when(k >= TNBUF)
                def _():
                    pltpu.make_async_copy(
                        obs[p], out_hbm.at[pl.ds(0, CBLK * D)],
                        souts[p]).wait()

                transpose(ins[p], obs[p])
                c0 = (wid + k * NUM_WORKERS) * CBLK
                pltpu.async_copy(
                    obs[p], out_hbm.at[pl.ds(c0 * D, CBLK * D)], souts[p])
        return carry

    lax.fori_loop(0, nrounds, rnd, 0)

    for p in range(TNBUF):
        @pl.when(p < nb)
        def _():
            pltpu.make_async_copy(
                obs[p], out_hbm.at[pl.ds(0, CBLK * D)], souts[p]).wait()

    if rem:
        @pl.when(wid == nfull % NUM_WORKERS)
        def _():
            pltpu.sync_copy(tail_hbm,
                            out_hbm.at[pl.ds(nfull * CBLK * D, rem * D)])


def _gather_kernel(batch, ids_hbm, table_hbm, out_hbm,
                   idx_v, acc0, acc1, out_v, sem0, sem1):
    bags_per_w = batch // NUM_WORKERS
    units_per_w = bags_per_w // UNIT_BAGS

    wid = lax.axis_index("s") * 2 + lax.axis_index("c")
    unit0 = wid * units_per_w
    bag0 = wid * bags_per_w

    pltpu.sync_copy(ids_hbm.at[:, pl.ds(unit0, units_per_w), :], idx_v)

    accs = (acc0, acc1)
    sems = (sem0, sem1)

    inv = jnp.float32(1.0 / SEQ)
    lo = pl.ds(0, 16)
    hi = pl.ds(16, 16)
    zvec = jnp.zeros((16,), jnp.float32)

    def zero(acc):
        def body(r, c):
            acc[r, lo] = zvec
            acc[r, hi] = zvec
            return c
        lax.fori_loop(0, UNIT_BAGS, body, 0)

    def fire(u, acc, sem):
        def step(st, c):
            for j in range(10):
                pltpu.async_copy(
                    table_hbm.at[idx_v.at[st * 10 + j, u]], acc, sem, add=True)
            return c
        lax.fori_loop(0, SEQ // 10, step, 0)

    def drain(acc, sem):
        def body(i, c):
            pltpu.make_async_copy(
                table_hbm.at[pl.ds(0, UNIT_BAGS)], acc, sem).wait()
            return c
        lax.fori_loop(0, SEQ, body, 0)

    def scale_out(u, acc):
        def body(r, c):
            bag = u * UNIT_BAGS + r
            out_v[bag, lo] = acc[r, lo] * inv
            out_v[bag, hi] = acc[r, hi] * inv
            return c
        lax.fori_loop(0, UNIT_BAGS, body, 0)

    for p in range(NBUF):
        zero(accs[p])
        fire(p, accs[p], sems[p])

    for u in range(units_per_w):
        p = u % NBUF
        drain(accs[p], sems[p])
        scale_out(u, accs[p])
        if u + NBUF < units_per_w:
            zero(accs[p])
            fire(u + NBUF, accs[p], sems[p])

    pltpu.sync_copy(out_v, out_hbm.at[pl.ds(bag0, bags_per_w)])


_SC_MESH = dict(core_axis_name="c", subcore_axis_name="s")


@jax.jit
def _bag_of_words(ids3, t32, tail_flat):
    vocab = t32.shape[1]
    batch = ids3.shape[1] * ids3.shape[2]
    bags_per_w = batch // NUM_WORKERS
    units_per_w = bags_per_w // UNIT_BAGS

    transpose_call = functools.partial(
        pl.kernel,
        mesh=plsc.VectorSubcoreMesh(**_SC_MESH),
        out_type=jax.ShapeDtypeStruct((vocab * D,), jnp.float32),
        scratch_types=(
            [pltpu.VMEM((D, CBLK), jnp.float32)] * TNBUF
            + [pltpu.VMEM((CBLK * D,), jnp.float32)] * TNBUF
            + [pltpu.SemaphoreType.DMA] * (2 * TNBUF)
        ),
        compiler_params=pltpu.CompilerParams(
            use_tc_tiling_on_sc=True, needs_layout_passes=False),
    )
    table_flat = transpose_call(
        functools.partial(_transpose_kernel, vocab))(t32, tail_flat)
    table_lin = table_flat.reshape(vocab, D)

    gather_call = functools.partial(
        pl.kernel,
        mesh=plsc.VectorSubcoreMesh(**_SC_MESH),
        out_type=jax.ShapeDtypeStruct((batch, D), jnp.float32),
        scratch_types=[
            pltpu.VMEM((SEQ, units_per_w, UNIT_BAGS), jnp.int32),
            pltpu.VMEM((UNIT_BAGS, D), jnp.float32),
            pltpu.VMEM((UNIT_BAGS, D), jnp.float32),
            pltpu.VMEM((bags_per_w, D), jnp.float32),
            pltpu.SemaphoreType.DMA,
            pltpu.SemaphoreType.DMA,
        ],
        compiler_params=pltpu.CompilerParams(use_tc_tiling_on_sc=False),
    )
    return gather_call(functools.partial(_gather_kernel, batch))(ids3, table_lin)


def kernel(token_ids, table):
    batch, seq = token_ids.shape
    assert seq == SEQ and table.shape[1] == D
    assert batch % (NUM_WORKERS * UNIT_BAGS) == 0
    ids3 = token_ids.astype(jnp.int32).T.reshape(
        SEQ, batch // UNIT_BAGS, UNIT_BAGS)
    vocab = table.shape[0]
    tail_flat = table[(vocab // CBLK) * CBLK:, :].reshape(-1)
    return _bag_of_words(ids3, table.T, tail_flat)


# speedup vs baseline: 1.0647x; 1.0647x over previous
import functools

import jax
import jax.numpy as jnp
from jax import lax
from jax.experimental import pallas as pl
from jax.experimental.pallas import tpu as pltpu, tpu_sc as plsc

D = 32
SEQ = 50
NUM_WORKERS = 32
UNIT_BAGS = 128
NBUF = 2
TNBUF = 3
CBLK = 512


def _transpose_kernel(vocab, t32_hbm, tail_hbm, out_hbm,
                      in0, in1, in2, ob0, ob1, ob2,
                      sin0, sin1, sin2, sout0, sout1, sout2):
    nfull = vocab // CBLK
    rem = vocab - nfull * CBLK
    maxnb = (nfull + NUM_WORKERS - 1) // NUM_WORKERS
    nrounds = (maxnb + TNBUF - 1) // TNBUF

    wid = lax.axis_index("s") * 2 + lax.axis_index("c")
    nb = (nfull - wid + NUM_WORKERS - 1) // NUM_WORKERS

    ins = (in0, in1, in2)
    obs = (ob0, ob1, ob2)
    sins = (sin0, sin1, sin2)
    souts = (sout0, sout1, sout2)

    iota = lax.iota(jnp.int32, 16)

    def fire_in(k, p):
        c0 = (wid + k * NUM_WORKERS) * CBLK
        pltpu.async_copy(t32_hbm.at[:, pl.ds(c0, CBLK)], ins[p], sins[p])

    def transpose(in_ref, ob_ref):
        @plsc.parallel_loop(0, 16, unroll=4)
        def tbody(t):
            dvec = (iota + t) & 15
            svec = iota * D + dvec
            for dg in range(D // 16):
                gd = dvec + dg * 16
                for cg in range(CBLK // 16):
                    v = plsc.load_gather(in_ref, [gd, iota + cg * 16])
                    plsc.store_scatter(
                        ob_ref, [svec + (cg * 16 * D + dg * 16)], v)

    for p in range(TNBUF - 1):
        @pl.when(p < nb)
        def _():
            fire_in(p, p)

    def rnd(kr, carry):
        for p in range(TNBUF):
            k = kr * TNBUF + p

            @pl.when(k < nb)
            def _():
                pltpu.make_async_copy(
                    t32_hbm.at[:, pl.ds(0, CBLK)], ins[p], sins[p]).wait()

                @pl.when(k + TNBUF - 1 < nb)
                def _():
                    fire_in(k + TNBUF - 1, (p + TNBUF - 1) % TNBUF)

                @pl.when(k >= TNBUF)
                def _():
                    pltpu.make_async_copy(
                        obs[p], out_hbm.at[pl.ds(0, CBLK * D)],
                        souts[p]).wait()

                transpose(ins[p], obs[p])
                c0 = (wid + k * NUM_WORKERS) * CBLK
                pltpu.async_copy(
                    obs[p], out_hbm.at[pl.ds(c0 * D, CBLK * D)], souts[p])
        return carry

    lax.fori_loop(0, nrounds, rnd, 0)

    for p in range(TNBUF):
        @pl.when(p < nb)
        def _():
            pltpu.make_async_copy(
                obs[p], out_hbm.at[pl.ds(0, CBLK * D)], souts[p]).wait()

    if rem:
        @pl.when(wid == nfull % NUM_WORKERS)
        def _():
            pltpu.sync_copy(tail_hbm,
                            out_hbm.at[pl.ds(nfull * CBLK * D, rem * D)])


def _gather_kernel(batch, ids_hbm, table_hbm, out_hbm,
                   idx_v, acc0, acc1, out_v, sem0, sem1):
    bags_per_w = batch // NUM_WORKERS
    units_per_w = bags_per_w // UNIT_BAGS

    wid = lax.axis_index("s") * 2 + lax.axis_index("c")
    unit0 = wid * units_per_w
    bag0 = wid * bags_per_w

    pltpu.sync_copy(ids_hbm.at[:, pl.ds(unit0, units_per_w), :], idx_v)

    accs = (acc0, acc1)
    sems = (sem0, sem1)

    inv = jnp.float32(1.0 / SEQ)
    lo = pl.ds(0, 16)
    hi = pl.ds(16, 16)
    zvec = jnp.zeros((16,), jnp.float32)

    def zero(acc):
        def body(r, c):
            acc[r, lo] = zvec
            acc[r, hi] = zvec
            return c
        lax.fori_loop(0, UNIT_BAGS, body, 0)

    def fire(u, acc, sem):
        def step(st, c):
            for j in range(10):
                pltpu.async_copy(
                    table_hbm.at[idx_v.at[st * 10 + j, u]], acc, sem, add=True)
            return c
        lax.fori_loop(0, SEQ // 10, step, 0)

    def drain(acc, sem):
        def body(i, c):
            pltpu.make_async_copy(
                table_hbm.at[pl.ds(0, UNIT_BAGS)], acc, sem).wait()
            return c
        lax.fori_loop(0, SEQ, body, 0)

    def scale_out(u, acc):
        def body(r, c):
            bag = u * UNIT_BAGS + r
            out_v[bag, lo] = acc[r, lo] * inv
            out_v[bag, hi] = acc[r, hi] * inv
            return c
        lax.fori_loop(0, UNIT_BAGS, body, 0)

    for p in range(NBUF):
        zero(accs[p])
        fire(p, accs[p], sems[p])

    for u in range(units_per_w):
        p = u % NBUF
        drain(accs[p], sems[p])
        scale_out(u, accs[p])
        if u + NBUF < units_per_w:
            zero(accs[p])
            fire(u + NBUF, accs[p], sems[p])

    pltpu.sync_copy(out_v, out_hbm.at[pl.ds(bag0, bags_per_w)])


_SC_MESH = dict(core_axis_name="c", subcore_axis_name="s")


@jax.jit
def _bag_of_words(ids3, t32, tail_flat):
    vocab = t32.shape[1]
    batch = ids3.shape[1] * ids3.shape[2]
    bags_per_w = batch // NUM_WORKERS
    units_per_w = bags_per_w // UNIT_BAGS

    transpose_call = functools.partial(
        pl.kernel,
        mesh=plsc.VectorSubcoreMesh(**_SC_MESH),
        out_type=jax.ShapeDtypeStruct((vocab * D,), jnp.float32),
        scratch_types=(
            [pltpu.VMEM((D, CBLK), jnp.float32)] * TNBUF
            + [pltpu.VMEM((CBLK * D,), jnp.float32)] * TNBUF
            + [pltpu.SemaphoreType.DMA] * (2 * TNBUF)
        ),
        compiler_params=pltpu.CompilerParams(
            use_tc_tiling_on_sc=True, needs_layout_passes=False),
    )
    table_flat = transpose_call(
        functools.partial(_transpose_kernel, vocab))(t32, tail_flat)
    table_lin = table_flat.reshape(vocab, D)

    gather_call = functools.partial(
        pl.kernel,
        mesh=plsc.VectorSubcoreMesh(**_SC_MESH),
        out_type=jax.ShapeDtypeStruct((batch, D), jnp.float32),
        scratch_types=[
            pltpu.VMEM((SEQ, units_per_w, UNIT_BAGS), jnp.int32),
            pltpu.VMEM((UNIT_BAGS, D), jnp.float32),
            pltpu.VMEM((UNIT_BAGS, D), jnp.float32),
            pltpu.VMEM((bags_per_w, D), jnp.float32),
            pltpu.SemaphoreType.DMA,
            pltpu.SemaphoreType.DMA,
        ],
        compiler_params=pltpu.CompilerParams(use_tc_tiling_on_sc=False),
    )
    return gather_call(functools.partial(_gather_kernel, batch))(ids3, table_lin)


def kernel(token_ids, table):
    batch, seq = token_ids.shape
    assert seq == SEQ and table.shape[1] == D
    assert batch % (NUM_WORKERS * UNIT_BAGS) == 0
    ids3 = token_ids.astype(jnp.int32).T.reshape(
        SEQ, batch // UNIT_BAGS, UNIT_BAGS)
    vocab = table.shape[0]
    tail_flat = table[(vocab // CBLK) * CBLK:, :].reshape(-1)
    return _bag_of_words(ids3, table.T, tail_flat)
